# Initial kernel scaffold; baseline (speedup 1.0000x reference)
import jax
import jax.numpy as jnp
from jax.experimental import pallas as pl


def kernel(x_author, edge_index, W_proj, b_proj, paper_emb, W1l_ap, b1l_ap, W1r_ap, W1l_pa, b1l_pa, W1r_pa, W2l_ap, b2l_ap, W2r_ap, W2l_pa, b2l_pa, W2r_pa, W_lin1, b_lin1, W_lin2, b_lin2):
    raise NotImplementedError("write your pallas kernel here")


# speedup vs baseline: 4.8946x; 4.8946x over previous
import functools

import jax
import jax.numpy as jnp
from jax import lax
from jax.experimental import pallas as pl
from jax.experimental.pallas import tpu as pltpu
from jax.experimental.pallas import tpu_sc as plsc

NA = 50000
NP = 50000
NE = 800000
DIN = 128
H = 64
HH = 32

NC, NS = 2, 16
ACC_N = 50048
RPT = ACC_N // NS
SUB = 128
NEP = 819200
EROWS = NEP // SUB
RPT_E = EROWS // NS
KSUB = 4
CH = KSUB * SUB
NCHUNK = RPT_E // KSUB
PAD_SEG = 50000

_f32 = jnp.float32


def _sc_mesh():
    return plsc.VectorSubcoreMesh(
        core_axis_name="c", subcore_axis_name="s", num_cores=NC, num_subcores=NS)


def _seg_passes(tables, gidx2d_list, sidx2d_list, deg_cores, z2d, z1d, ones1):
    n_pass = len(tables)
    out_type = [jax.ShapeDtypeStruct((NC, ACC_N, HH), _f32) for _ in range(n_pass)]
    n_deg = sum(1 for d in deg_cores if d >= 0)
    out_type += [jax.ShapeDtypeStruct((ACC_N,), _f32) for _ in range(n_deg)]

    @functools.partial(
        pl.kernel,
        out_type=tuple(out_type),
        mesh=_sc_mesh(),
        compiler_params=pltpu.CompilerParams(use_tc_tiling_on_sc=False),
        scratch_types=[
            pltpu.VMEM_SHARED((ACC_N, HH), _f32),
            pltpu.VMEM_SHARED((ACC_N,), _f32),
            pltpu.VMEM((KSUB, SUB), jnp.int32),
            pltpu.VMEM((KSUB, SUB), jnp.int32),
            pltpu.VMEM((CH, HH), _f32),
            pltpu.VMEM((SUB,), _f32),
            pltpu.SemaphoreType.DMA,
        ],
    )
    def sc_kernel(*refs):
        ins = refs[:3 * n_pass + 3]
        outs = refs[3 * n_pass + 3:3 * n_pass + 3 + n_pass + n_deg]
        acc, cnt, gidx, sidx, rows, onev, sem = refs[3 * n_pass + 3 + n_pass + n_deg:]
        tab_refs = ins[0:n_pass]
        g_refs = ins[n_pass:2 * n_pass]
        s_refs = ins[2 * n_pass:3 * n_pass]
        z2_hbm, z1_hbm, one_hbm = ins[3 * n_pass:3 * n_pass + 3]
        sum_outs = outs[:n_pass]
        deg_outs = outs[n_pass:]

        c = lax.axis_index("c")
        s = lax.axis_index("s")
        r0 = s * RPT
        pltpu.sync_copy(one_hbm, onev)

        deg_i = 0
        for p in range(n_pass):
            tab_hbm, g_hbm, s_hbm = tab_refs[p], g_refs[p], s_refs[p]
            out_hbm = sum_outs[p]
            deg_core = deg_cores[p]
            do_deg = deg_core >= 0
            deg_out = deg_outs[deg_i] if do_deg else None
            if do_deg:
                deg_i += 1

            pltpu.sync_copy(z2_hbm, acc.at[pl.ds(r0, RPT)])
            if do_deg:
                pltpu.sync_copy(z1_hbm, cnt.at[pl.ds(r0, RPT)])
            plsc.subcore_barrier()

            def chunk(j, carry):
                row0 = s * RPT_E + j * KSUB
                pltpu.sync_copy(g_hbm.at[pl.ds(row0, KSUB)], gidx)
                pltpu.sync_copy(s_hbm.at[pl.ds(row0, KSUB)], sidx)
                cps = [
                    pltpu.async_copy(
                        tab_hbm.at[c].at[gidx.at[kk]],
                        rows.at[pl.ds(kk * SUB, SUB)], sem)
                    for kk in range(KSUB)
                ]
                for cp in cps:
                    cp.wait()
                for kk in range(KSUB):
                    pltpu.sync_copy(rows.at[pl.ds(kk * SUB, SUB)],
                                    acc.at[sidx.at[kk]], add=True)
                if do_deg:
                    @pl.when(c == deg_core)
                    def _():
                        for kk in range(KSUB):
                            pltpu.sync_copy(onev, cnt.at[sidx.at[kk]], add=True)
                return carry

            lax.fori_loop(0, NCHUNK, chunk, 0)
            plsc.subcore_barrier()

            pltpu.sync_copy(acc.at[pl.ds(r0, RPT)],
                            out_hbm.at[c].at[pl.ds(r0, RPT)])
            if do_deg:
                @pl.when(c == deg_core)
                def _():
                    pltpu.sync_copy(cnt.at[pl.ds(r0, RPT)],
                                    deg_out.at[pl.ds(r0, RPT)])
            plsc.subcore_barrier()

    args = list(tables) + list(gidx2d_list) + list(sidx2d_list) + [z2d, z1d, ones1]
    return sc_kernel(*args)


_R = 1000
_G = NA // _R


def _tc_project(x_author, W_proj, b_proj, paper_emb):
    def body(x_ref, w_ref, b_ref, pe_ref, xa_ref, xp_ref):
        xa = jnp.dot(x_ref[...], w_ref[...],
                     preferred_element_type=_f32) + b_ref[...]
        xa_ref[0] = xa[:, :HH]
        xa_ref[1] = xa[:, HH:]
        pe = pe_ref[...]
        xp_ref[0] = pe[:, :HH]
        xp_ref[1] = pe[:, HH:]

    return pl.pallas_call(
        body,
        grid=(_G,),
        in_specs=[
            pl.BlockSpec((_R, DIN), lambda i: (i, 0)),
            pl.BlockSpec((DIN, H), lambda i: (0, 0)),
            pl.BlockSpec((1, H), lambda i: (0, 0)),
            pl.BlockSpec((_R, H), lambda i: (i, 0)),
        ],
        out_specs=[
            pl.BlockSpec((NC, _R, HH), lambda i: (0, i, 0)),
            pl.BlockSpec((NC, _R, HH), lambda i: (0, i, 0)),
        ],
        out_shape=[
            jax.ShapeDtypeStruct((NC, NA, HH), _f32),
            jax.ShapeDtypeStruct((NC, NP, HH), _f32),
        ],
    )(x_author, W_proj, b_proj.reshape(1, H), paper_emb)


def _tc_conv1(s1, s2, degp, dega, xa_st, xp_st,
              W1l_ap, b1l_ap, W1r_ap, W1l_pa, b1l_pa, W1r_pa):
    def body(s1_ref, s2_ref, dp_ref, da_ref, xa_ref, xp_ref,
             wlap_ref, blap_ref, wrap_ref, wlpa_ref, blpa_ref, wrpa_ref,
             p1_ref, a1_ref):
        s1b = jnp.concatenate([s1_ref[0], s1_ref[1]], axis=1)
        m1 = s1b / jnp.maximum(dp_ref[...], 1.0)
        xp = jnp.concatenate([xp_ref[0], xp_ref[1]], axis=1)
        p1 = jax.nn.relu(
            jnp.dot(m1, wlap_ref[...], preferred_element_type=_f32)
            + blap_ref[...]
            + jnp.dot(xp, wrap_ref[...], preferred_element_type=_f32))
        p1_ref[...] = p1
        s2b = jnp.concatenate([s2_ref[0], s2_ref[1]], axis=1)
        m2 = s2b / jnp.maximum(da_ref[...], 1.0)
        xa = jnp.concatenate([xa_ref[0], xa_ref[1]], axis=1)
        a1 = jax.nn.relu(
            jnp.dot(m2, wlpa_ref[...], preferred_element_type=_f32)
            + blpa_ref[...]
            + jnp.dot(xa, wrpa_ref[...], preferred_element_type=_f32))
        a1_ref[0] = a1[:, :HH]
        a1_ref[1] = a1[:, HH:]

    half = lambda: pl.BlockSpec((NC, _R, HH), lambda i: (0, i, 0))
    col = lambda: pl.BlockSpec((_R, 1), lambda i: (i, 0))
    wmat = lambda: pl.BlockSpec((H, H), lambda i: (0, 0))
    wvec = lambda: pl.BlockSpec((1, H), lambda i: (0, 0))
    return pl.pallas_call(
        body,
        grid=(_G,),
        in_specs=[half(), half(), col(), col(), half(), half(),
                  wmat(), wvec(), wmat(), wmat(), wvec(), wmat()],
        out_specs=[
            pl.BlockSpec((_R, H), lambda i: (i, 0)),
            pl.BlockSpec((NC, _R, HH), lambda i: (0, i, 0)),
        ],
        out_shape=[
            jax.ShapeDtypeStruct((NP, H), _f32),
            jax.ShapeDtypeStruct((NC, NA, HH), _f32),
        ],
    )(s1, s2, degp, dega, xa_st, xp_st,
      W1l_ap, b1l_ap.reshape(1, H), W1r_ap,
      W1l_pa, b1l_pa.reshape(1, H), W1r_pa)


def _tc_conv2_head(s3, degp, p1, W2l_ap, b2l_ap, W2r_ap,
                   W_lin1, b_lin1, W_lin2, b_lin2):
    def body(s3_ref, dp_ref, p1_ref, wl_ref, bl_ref, wr_ref,
             w1_ref, b1_ref, w2_ref, b2_ref, o_ref):
        s3b = jnp.concatenate([s3_ref[0], s3_ref[1]], axis=1)
        m3 = s3b / jnp.maximum(dp_ref[...], 1.0)
        p2 = jax.nn.relu(
            jnp.dot(m3, wl_ref[...], preferred_element_type=_f32)
            + bl_ref[...]
            + jnp.dot(p1_ref[...], wr_ref[...], preferred_element_type=_f32))
        h = jax.nn.relu(
            jnp.dot(p2, w1_ref[...], preferred_element_type=_f32) + b1_ref[...])
        o_ref[...] = jnp.dot(h, w2_ref[...],
                             preferred_element_type=_f32) + b2_ref[...]

    return pl.pallas_call(
        body,
        grid=(_G,),
        in_specs=[
            pl.BlockSpec((NC, _R, HH), lambda i: (0, i, 0)),
            pl.BlockSpec((_R, 1), lambda i: (i, 0)),
            pl.BlockSpec((_R, H), lambda i: (i, 0)),
            pl.BlockSpec((H, H), lambda i: (0, 0)),
            pl.BlockSpec((1, H), lambda i: (0, 0)),
            pl.BlockSpec((H, H), lambda i: (0, 0)),
            pl.BlockSpec((H, H), lambda i: (0, 0)),
            pl.BlockSpec((1, H), lambda i: (0, 0)),
            pl.BlockSpec((H, 1), lambda i: (0, 0)),
            pl.BlockSpec((1, 1), lambda i: (0, 0)),
        ],
        out_specs=pl.BlockSpec((_R, 1), lambda i: (i, 0)),
        out_shape=jax.ShapeDtypeStruct((NP, 1), _f32),
    )(s3, degp, p1, W2l_ap, b2l_ap.reshape(1, H), W2r_ap,
      W_lin1, b_lin1.reshape(1, H), W_lin2, b_lin2.reshape(1, 1))


def kernel(x_author, edge_index, W_proj, b_proj, paper_emb,
           W1l_ap, b1l_ap, W1r_ap, W1l_pa, b1l_pa, W1r_pa,
           W2l_ap, b2l_ap, W2r_ap, W2l_pa, b2l_pa, W2r_pa,
           W_lin1, b_lin1, W_lin2, b_lin2):
    src = edge_index[0].astype(jnp.int32)
    dst = edge_index[1].astype(jnp.int32)
    pad = NEP - NE
    src2d = jnp.concatenate([src, jnp.zeros((pad,), jnp.int32)]).reshape(EROWS, SUB)
    dst2d = jnp.concatenate(
        [dst, jnp.full((pad,), PAD_SEG, jnp.int32)]).reshape(EROWS, SUB)
    z2d = jnp.zeros((RPT, HH), _f32)
    z1d = jnp.zeros((RPT,), _f32)
    ones1 = jnp.ones((SUB,), _f32)

    xa_st, xp_st = _tc_project(x_author, W_proj, b_proj, paper_emb)

    s1, s2, degp, dega = _seg_passes(
        [xa_st, xp_st], [src2d, dst2d], [dst2d, src2d], [0, 1],
        z2d, z1d, ones1)

    degp = degp.reshape(ACC_N, 1)
    dega = dega.reshape(ACC_N, 1)
    p1, a1_st = _tc_conv1(s1, s2, degp, dega, xa_st, xp_st,
                          W1l_ap, b1l_ap, W1r_ap, W1l_pa, b1l_pa, W1r_pa)

    (s3,) = _seg_passes([a1_st], [src2d], [dst2d], [-1], z2d, z1d, ones1)

    return _tc_conv2_head(s3, degp, p1, W2l_ap, b2l_ap, W2r_ap,
                          W_lin1, b_lin1, W_lin2, b_lin2)


# speedup vs baseline: 6.1188x; 1.2501x over previous
import functools

import jax
import jax.numpy as jnp
from jax import lax
from jax.experimental import pallas as pl
from jax.experimental.pallas import tpu as pltpu
from jax.experimental.pallas import tpu_sc as plsc

NA = 50000
NP = 50000
NE = 800000
DIN = 128
H = 64
HH = 32

NC, NS = 2, 16
ACC_N = 50048
RPT = ACC_N // NS
SUB = 128
NEP = 819200
EROWS = NEP // SUB
RPT_E = EROWS // NS
KS = 8
NIT = RPT_E // KS
NIT2 = NIT // 2
NBUF = 4
PAD_SEG = 50000

_f32 = jnp.float32


def _sc_mesh():
    return plsc.VectorSubcoreMesh(
        core_axis_name="c", subcore_axis_name="s", num_cores=NC, num_subcores=NS)


def _seg_passes(tables, g_list, s_list, deg_pass0, z2d, z1d, ones1):
    n_pass = len(tables)
    n_deg = 2 if deg_pass0 else 0
    out_type = [jax.ShapeDtypeStruct((NC, ACC_N, HH), _f32) for _ in range(n_pass)]
    out_type += [jax.ShapeDtypeStruct((ACC_N,), _f32)] * n_deg

    @functools.partial(
        pl.kernel,
        out_type=tuple(out_type),
        mesh=_sc_mesh(),
        compiler_params=pltpu.CompilerParams(use_tc_tiling_on_sc=False),
        scratch_types=[
            pltpu.VMEM_SHARED((ACC_N, HH), _f32),
            pltpu.VMEM_SHARED((ACC_N,), _f32),
            pltpu.VMEM((2, KS, SUB), jnp.int32),
            pltpu.VMEM((2, KS, SUB), jnp.int32),
        ] + [pltpu.VMEM((SUB, HH), _f32) for _ in range(NBUF)]
        + [pltpu.VMEM((SUB,), _f32)]
        + [pltpu.SemaphoreType.DMA] * (2 * NBUF + 3),
    )
    def sc_kernel(*refs):
        ins = refs[:3 * n_pass + 3]
        outs = refs[3 * n_pass + 3:3 * n_pass + 3 + n_pass + n_deg]
        scr = refs[3 * n_pass + 3 + n_pass + n_deg:]
        tab_refs = ins[0:n_pass]
        g_refs = ins[n_pass:2 * n_pass]
        s_refs = ins[2 * n_pass:3 * n_pass]
        z2_hbm, z1_hbm, one_hbm = ins[3 * n_pass:3 * n_pass + 3]
        sum_outs = outs[:n_pass]
        deg_outs = outs[n_pass:]
        acc, cnt, gidx, sidx = scr[0:4]
        rows = scr[4:4 + NBUF]
        onev = scr[4 + NBUF]
        sems = scr[5 + NBUF:]
        semG = sems[0:NBUF]
        semS = sems[NBUF:2 * NBUF]
        semI = sems[2 * NBUF:2 * NBUF + 2]
        semC = sems[2 * NBUF + 2]

        c = lax.axis_index("c")
        s = lax.axis_index("s")
        r0 = s * RPT
        e_base = s * RPT_E
        pltpu.sync_copy(one_hbm, onev)

        def drain_scatter(b):
            pltpu.make_async_copy(
                z2_hbm.at[pl.ds(0, SUB)], rows[b], semS[b]).wait()

        def drain_cnt():
            pltpu.make_async_copy(
                z1_hbm.at[pl.ds(0, SUB)], onev, semC).wait()

        def run_pass(tab_hbm, g_hbm, s_hbm, out_hbm, do_deg):
            pltpu.sync_copy(z2_hbm, acc.at[pl.ds(r0, RPT)])
            if do_deg:
                pltpu.sync_copy(z1_hbm, cnt.at[pl.ds(r0, RPT)])
            plsc.subcore_barrier()

            def idx_fire(j, sl):
                row0 = e_base + j * KS
                pltpu.async_copy(g_hbm.at[pl.ds(row0, KS)], gidx.at[sl], semI[sl])
                pltpu.async_copy(s_hbm.at[pl.ds(row0, KS)], sidx.at[sl], semI[sl])

            def idx_wait(sl):
                for _ in range(2):
                    pltpu.make_async_copy(
                        g_hbm.at[pl.ds(e_base, KS)], gidx.at[sl], semI[sl]).wait()

            pltpu.sync_copy(g_hbm.at[pl.ds(e_base, KS)], gidx.at[0])
            pltpu.sync_copy(s_hbm.at[pl.ds(e_base, KS)], sidx.at[0])

            def one_iter(j2, jj):
                j = j2 * 2 + jj
                sl = jj
                not_first = (j2 > 0) if jj == 0 else None

                def guarded(body):
                    if not_first is None:
                        body()
                    else:
                        pl.when(not_first)(body)

                guarded(lambda: [drain_scatter(b) for b in range(NBUF)] and None)
                if do_deg:
                    guarded(lambda: [drain_cnt() for _ in range(KS)] and None)
                guarded(lambda: idx_wait(sl))
                if jj == 0:
                    idx_fire(j + 1, 1)
                else:
                    pl.when(j2 < NIT2 - 1)(lambda: idx_fire(j + 1, 0))
                for k in range(KS + 2):
                    if k < KS:
                        b = k % NBUF
                        if k >= NBUF:
                            drain_scatter(b)
                        pltpu.async_copy(
                            tab_hbm.at[c].at[gidx.at[sl, k]], rows[b], semG[b])
                    kr = k - 2
                    if kr >= 0:
                        br = kr % NBUF
                        pltpu.make_async_copy(
                            tab_hbm.at[c].at[gidx.at[sl, kr]], rows[br],
                            semG[br]).wait()
                        pltpu.async_copy(rows[br], acc.at[sidx.at[sl, kr]],
                                         semS[br], add=True)
                        if do_deg:
                            @pl.when(c == 0)
                            def _():
                                pltpu.async_copy(
                                    onev, cnt.at[sidx.at[sl, kr]], semC, add=True)
                            @pl.when(c == 1)
                            def _():
                                pltpu.async_copy(
                                    onev, cnt.at[gidx.at[sl, kr]], semC, add=True)
                return None

            def outer(j2, carry):
                one_iter(j2, 0)
                one_iter(j2, 1)
                return carry

            lax.fori_loop(0, NIT2, outer, 0)

            for b in range(NBUF):
                drain_scatter(b)
            if do_deg:
                for _ in range(KS):
                    drain_cnt()
            plsc.subcore_barrier()

            pltpu.sync_copy(acc.at[pl.ds(r0, RPT)],
                            out_hbm.at[c].at[pl.ds(r0, RPT)])
            if do_deg:
                @pl.when(c == 0)
                def _():
                    pltpu.sync_copy(cnt.at[pl.ds(r0, RPT)],
                                    deg_outs[0].at[pl.ds(r0, RPT)])
                @pl.when(c == 1)
                def _():
                    pltpu.sync_copy(cnt.at[pl.ds(r0, RPT)],
                                    deg_outs[1].at[pl.ds(r0, RPT)])
            plsc.subcore_barrier()

        for p in range(n_pass):
            run_pass(tab_refs[p], g_refs[p], s_refs[p], sum_outs[p],
                     deg_pass0 and p == 0)

    args = list(tables) + list(g_list) + list(s_list) + [z2d, z1d, ones1]
    return sc_kernel(*args)


_R = 1000
_G = NA // _R


def _tc_project(x_author, W_proj, b_proj, paper_emb):
    def body(x_ref, w_ref, b_ref, pe_ref, xa_ref, xp_ref):
        xa = jnp.dot(x_ref[...], w_ref[...],
                     preferred_element_type=_f32) + b_ref[...]
        xa_ref[0] = xa[:, :HH]
        xa_ref[1] = xa[:, HH:]
        pe = pe_ref[...]
        xp_ref[0] = pe[:, :HH]
        xp_ref[1] = pe[:, HH:]

    return pl.pallas_call(
        body,
        grid=(_G,),
        in_specs=[
            pl.BlockSpec((_R, DIN), lambda i: (i, 0)),
            pl.BlockSpec((DIN, H), lambda i: (0, 0)),
            pl.BlockSpec((1, H), lambda i: (0, 0)),
            pl.BlockSpec((_R, H), lambda i: (i, 0)),
        ],
        out_specs=[
            pl.BlockSpec((NC, _R, HH), lambda i: (0, i, 0)),
            pl.BlockSpec((NC, _R, HH), lambda i: (0, i, 0)),
        ],
        out_shape=[
            jax.ShapeDtypeStruct((NC, ACC_N, HH), _f32),
            jax.ShapeDtypeStruct((NC, ACC_N, HH), _f32),
        ],
    )(x_author, W_proj, b_proj.reshape(1, H), paper_emb)


def _tc_conv1(s1, s2, degp, dega, xa_st, xp_st,
              W1l_ap, b1l_ap, W1r_ap, W1l_pa, b1l_pa, W1r_pa):
    def body(s1_ref, s2_ref, dp_ref, da_ref, xa_ref, xp_ref,
             wlap_ref, blap_ref, wrap_ref, wlpa_ref, blpa_ref, wrpa_ref,
             p1_ref, a1_ref):
        s1b = jnp.concatenate([s1_ref[0], s1_ref[1]], axis=1)
        m1 = s1b / jnp.maximum(dp_ref[...], 1.0)
        xp = jnp.concatenate([xp_ref[0], xp_ref[1]], axis=1)
        p1 = jax.nn.relu(
            jnp.dot(m1, wlap_ref[...], preferred_element_type=_f32)
            + blap_ref[...]
            + jnp.dot(xp, wrap_ref[...], preferred_element_type=_f32))
        p1_ref[...] = p1
        s2b = jnp.concatenate([s2_ref[0], s2_ref[1]], axis=1)
        m2 = s2b / jnp.maximum(da_ref[...], 1.0)
        xa = jnp.concatenate([xa_ref[0], xa_ref[1]], axis=1)
        a1 = jax.nn.relu(
            jnp.dot(m2, wlpa_ref[...], preferred_element_type=_f32)
            + blpa_ref[...]
            + jnp.dot(xa, wrpa_ref[...], preferred_element_type=_f32))
        a1_ref[0] = a1[:, :HH]
        a1_ref[1] = a1[:, HH:]

    half = lambda: pl.BlockSpec((NC, _R, HH), lambda i: (0, i, 0))
    col = lambda: pl.BlockSpec((_R, 1), lambda i: (i, 0))
    wmat = lambda: pl.BlockSpec((H, H), lambda i: (0, 0))
    wvec = lambda: pl.BlockSpec((1, H), lambda i: (0, 0))
    return pl.pallas_call(
        body,
        grid=(_G,),
        in_specs=[half(), half(), col(), col(), half(), half(),
                  wmat(), wvec(), wmat(), wmat(), wvec(), wmat()],
        out_specs=[
            pl.BlockSpec((_R, H), lambda i: (i, 0)),
            pl.BlockSpec((NC, _R, HH), lambda i: (0, i, 0)),
        ],
        out_shape=[
            jax.ShapeDtypeStruct((NP, H), _f32),
            jax.ShapeDtypeStruct((NC, ACC_N, HH), _f32),
        ],
    )(s1, s2, degp, dega, xa_st, xp_st,
      W1l_ap, b1l_ap.reshape(1, H), W1r_ap,
      W1l_pa, b1l_pa.reshape(1, H), W1r_pa)


def _tc_conv2_head(s3, degp, p1, W2l_ap, b2l_ap, W2r_ap,
                   W_lin1, b_lin1, W_lin2, b_lin2):
    def body(s3_ref, dp_ref, p1_ref, wl_ref, bl_ref, wr_ref,
             w1_ref, b1_ref, w2_ref, b2_ref, o_ref):
        s3b = jnp.concatenate([s3_ref[0], s3_ref[1]], axis=1)
        m3 = s3b / jnp.maximum(dp_ref[...], 1.0)
        p2 = jax.nn.relu(
            jnp.dot(m3, wl_ref[...], preferred_element_type=_f32)
            + bl_ref[...]
            + jnp.dot(p1_ref[...], wr_ref[...], preferred_element_type=_f32))
        h = jax.nn.relu(
            jnp.dot(p2, w1_ref[...], preferred_element_type=_f32) + b1_ref[...])
        o_ref[...] = jnp.dot(h, w2_ref[...],
                             preferred_element_type=_f32) + b2_ref[...]

    return pl.pallas_call(
        body,
        grid=(_G,),
        in_specs=[
            pl.BlockSpec((NC, _R, HH), lambda i: (0, i, 0)),
            pl.BlockSpec((_R, 1), lambda i: (i, 0)),
            pl.BlockSpec((_R, H), lambda i: (i, 0)),
            pl.BlockSpec((H, H), lambda i: (0, 0)),
            pl.BlockSpec((1, H), lambda i: (0, 0)),
            pl.BlockSpec((H, H), lambda i: (0, 0)),
            pl.BlockSpec((H, H), lambda i: (0, 0)),
            pl.BlockSpec((1, H), lambda i: (0, 0)),
            pl.BlockSpec((H, 1), lambda i: (0, 0)),
            pl.BlockSpec((1, 1), lambda i: (0, 0)),
        ],
        out_specs=pl.BlockSpec((_R, 1), lambda i: (i, 0)),
        out_shape=jax.ShapeDtypeStruct((NP, 1), _f32),
    )(s3, degp, p1, W2l_ap, b2l_ap.reshape(1, H), W2r_ap,
      W_lin1, b_lin1.reshape(1, H), W_lin2, b_lin2.reshape(1, 1))


def kernel(x_author, edge_index, W_proj, b_proj, paper_emb,
           W1l_ap, b1l_ap, W1r_ap, W1l_pa, b1l_pa, W1r_pa,
           W2l_ap, b2l_ap, W2r_ap, W2l_pa, b2l_pa, W2r_pa,
           W_lin1, b_lin1, W_lin2, b_lin2):
    src = edge_index[0].astype(jnp.int32)
    dst = edge_index[1].astype(jnp.int32)
    padv = jnp.full((NEP - NE,), PAD_SEG, jnp.int32)
    src2d = jnp.concatenate([src, padv]).reshape(EROWS, SUB)
    dst2d = jnp.concatenate([dst, padv]).reshape(EROWS, SUB)
    z2d = jnp.zeros((RPT, HH), _f32)
    z1d = jnp.zeros((RPT,), _f32)
    ones1 = jnp.ones((SUB,), _f32)

    xa_st, xp_st = _tc_project(x_author, W_proj, b_proj, paper_emb)

    s1, s2, degp, dega = _seg_passes(
        [xa_st, xp_st], [src2d, dst2d], [dst2d, src2d], True,
        z2d, z1d, ones1)

    degp = degp.reshape(ACC_N, 1)
    dega = dega.reshape(ACC_N, 1)
    p1, a1_st = _tc_conv1(s1, s2, degp, dega, xa_st, xp_st,
                          W1l_ap, b1l_ap, W1r_ap, W1l_pa, b1l_pa, W1r_pa)

    (s3,) = _seg_passes([a1_st], [src2d], [dst2d], False, z2d, z1d, ones1)

    return _tc_conv2_head(s3, degp, p1, W2l_ap, b2l_ap, W2r_ap,
                          W_lin1, b_lin1, W_lin2, b_lin2)
